# baseline (speedup 1.0000x reference)
import functools

import jax
import jax.numpy as jnp
import numpy as np
from jax.experimental import pallas as pl
from jax.experimental.pallas import tpu as pltpu

N_HEADS, HEAD_DIM = 4, 32
HH = N_HEADS * HEAD_DIM
PAD = 32
PAST_OFF = [1, 2, 3, 4, 5]
FUT_OFF = [-1, -2, -3, -4, -5]
PER_OFF = [-15, 15]
ALL_OFF = [0] + PAST_OFF + FUT_OFF + PER_OFF
NTAP = len(ALL_OFF)
NTAP16 = 16
NEG = -1e30


def _lroll(x, s):
    return jnp.concatenate([x[:, s:], x[:, :s]], axis=1)


def _mega_kernel(x_ref, few_ref, feb_ref, feg_ref, febeta_ref,
                 catw0_ref, rb0_ref, catp0_ref, catpb0_ref, bg0_ref, bb0_ref,
                 catw1_ref, rb1_ref, catp1_ref, catpb1_ref, bg1_ref, bb1_ref,
                 inv_ref, abias_ref, hsel_ref, hexp_ref, densel_ref,
                 o_ref, s_nf, s_pre, s_sum0, s_sumsq0, s_sum1, s_sumsq1,
                 scr, *, n, blk, nb):
    s = pl.program_id(0)
    ext = blk + 2 * PAD
    mid0, nmid = PAD - 15, blk + 30

    @pl.when(s < nb)
    def _fe():
        n0 = s * blk
        xb = x_ref[...]
        c = few_ref.shape[0]
        feat = (xb[:, :c] + xb[:, c:2 * c] + xb[:, 2 * c:3 * c]
                + xb[:, 3 * c:4 * c]) * 0.25
        nf = jnp.dot(feat, few_ref[...], preferred_element_type=jnp.float32) + feb_ref[...]
        mu = jnp.mean(nf, axis=-1, keepdims=True)
        var = jnp.mean((nf - mu) ** 2, axis=-1, keepdims=True)
        y = (nf - mu) * jax.lax.rsqrt(var + 1e-5) * feg_ref[...] + febeta_ref[...]

        @pl.when(s == 0)
        def _zero_pads():
            z = jnp.zeros((PAD, HH), jnp.float32)
            s_nf[0:PAD, :] = z
            s_nf[PAD + n:2 * PAD + n, :] = z
            s_pre[0:PAD, :] = z
            s_pre[PAD + n:2 * PAD + n, :] = z

        s_nf[pl.ds(PAD + n0, blk), :] = y

    def _layer_body(j, xe, catw_ref, rb_ref, catp_ref, catpb_ref,
                    dst, sum_ref, sumsq_ref):
        n0 = j * blk
        x_mid = xe[mid0:mid0 + nmid]
        u1 = xe[mid0 + 1:mid0 + 1 + nmid]
        u2 = xe[mid0 - 1:mid0 - 1 + nmid]
        for o in PAST_OFF[1:]:
            u1 = u1 + xe[mid0 + o:mid0 + o + nmid]
            u2 = u2 + xe[mid0 - o:mid0 - o + nmid]
        u3 = xe[mid0 - 15:mid0 - 15 + nmid] + xe[mid0 + 15:mid0 + 15 + nmid]
        inv = inv_ref[pl.ds(n0 + mid0, nmid), :]
        cat = jnp.concatenate(
            [x_mid, u1 * inv[:, 0:1], u2 * inv[:, 1:2], u3 * inv[:, 2:3]], axis=1)
        h = jnp.dot(cat, catw_ref[...], preferred_element_type=jnp.float32) + rb_ref[...]

        qkvs = jnp.dot(h, catp_ref[...], preferred_element_type=jnp.float32) + catpb_ref[...]
        q = qkvs[15:15 + blk, 0:HH]
        k = qkvs[:, HH:2 * HH]
        v = qkvs[:, 2 * HH:3 * HH]

        hsel = hsel_ref[...]
        for t, d in enumerate(ALL_OFF):
            kd = k[15 + d:15 + d + blk]
            scr[:, 4 * t:4 * t + 4] = jnp.dot(q * kd, hsel,
                                              preferred_element_type=jnp.float32)
        scr[:, 4 * NTAP:4 * NTAP16] = jnp.zeros((blk, 4 * (NTAP16 - NTAP)), jnp.float32)

        a64 = scr[...] + abias_ref[pl.ds(n0, blk), :]
        m = jnp.maximum(a64, _lroll(a64, 32))
        m = jnp.maximum(m, _lroll(m, 16))
        m = jnp.maximum(m, _lroll(m, 8))
        m = jnp.maximum(m, _lroll(m, 4))
        ex = jnp.exp(a64 - m)
        denb = jnp.dot(ex, densel_ref[...], preferred_element_type=jnp.float32)

        hexp = hexp_ref[...]
        num = jnp.zeros((blk, HH), jnp.float32)
        for t, d in enumerate(ALL_OFF):
            exb = jnp.dot(ex[:, 4 * t:4 * t + 4], hexp,
                          preferred_element_type=jnp.float32)
            num = num + exb * v[15 + d:15 + d + blk]

        out = num / jnp.maximum(denb, 1e-16) + qkvs[15:15 + blk, 3 * HH:4 * HH]
        dst[pl.ds(PAD + n0, blk), :] = out
        ps = jnp.sum(out, axis=0, keepdims=True)
        pq = jnp.sum(out * out, axis=0, keepdims=True)

        @pl.when(j == 0)
        def _init_stats():
            sum_ref[...] = ps
            sumsq_ref[...] = pq

        @pl.when(j > 0)
        def _acc_stats():
            sum_ref[...] = sum_ref[...] + ps
            sumsq_ref[...] = sumsq_ref[...] + pq

    def _bn(xw, sum_ref, sumsq_ref, bg_ref, bb_ref):
        mu = sum_ref[...] * (1.0 / n)
        var = sumsq_ref[...] * (1.0 / n) - mu * mu
        y = (xw - mu) * jax.lax.rsqrt(var + 1e-5) * bg_ref[...] + bb_ref[...]
        return jnp.where(y >= 0.0, y, 0.01 * y)

    @pl.when(jnp.logical_and(s >= 1, s <= nb))
    def _layer0():
        j = s - 1
        xe = s_nf[pl.ds(j * blk, ext), :]
        _layer_body(j, xe, catw0_ref, rb0_ref, catp0_ref, catpb0_ref,
                    s_pre, s_sum0, s_sumsq0)

    @pl.when(jnp.logical_and(s >= nb + 1, s <= 2 * nb))
    def _layer1():
        j = s - nb - 1
        n0 = j * blk
        raw = s_pre[pl.ds(n0, ext), :]
        vmask = inv_ref[pl.ds(n0, ext), 3:4]
        xe = _bn(raw, s_sum0, s_sumsq0, bg0_ref, bb0_ref) * vmask
        _layer_body(j, xe, catw1_ref, rb1_ref, catp1_ref, catpb1_ref,
                    s_nf, s_sum1, s_sumsq1)

    @pl.when(s >= 2 * nb + 1)
    def _bn1():
        j = s - 2 * nb - 1
        xb = s_nf[pl.ds(PAD + j * blk, blk), :]
        o_ref[...] = _bn(xb, s_sum1, s_sumsq1, bg1_ref, bb1_ref)


def _graph_consts(n):
    g = np.arange(n)
    deltas = np.asarray(ALL_OFF)
    valid = (g[:, None] + deltas[None, :] >= 0) & (g[:, None] + deltas[None, :] < n)
    abias13 = np.where(valid, 0.0, NEG).astype(np.float32)
    abias = np.full((n, 4 * NTAP16), NEG, np.float32)
    abias[:, :4 * NTAP] = np.repeat(abias13, 4, axis=1)

    gp = np.arange(-PAD, n + PAD).astype(np.float32)
    inv = np.zeros((n + 2 * PAD, 4), np.float32)
    inv[:, 0] = 1.0 / np.maximum(np.minimum(5.0, (n - 1) - gp), 1.0)
    inv[:, 1] = 1.0 / np.maximum(np.minimum(5.0, gp), 1.0)
    inv[:, 2] = 1.0 / np.maximum((gp >= 15).astype(np.float32)
                                 + (gp <= n - 16).astype(np.float32), 1.0)
    inv[:, 3] = ((gp >= 0) & (gp <= n - 1)).astype(np.float32)

    hsel = np.repeat(np.eye(N_HEADS, dtype=np.float32), HEAD_DIM, axis=0)
    hsel_s = hsel / np.sqrt(HEAD_DIM).astype(np.float32)
    hexp = hsel.T
    lanes = np.arange(4 * NTAP16)
    densel = (lanes[:, None] % 4 == np.arange(HH)[None, :] // HEAD_DIM).astype(np.float32)
    return (jnp.asarray(abias), jnp.asarray(inv), jnp.asarray(hsel_s),
            jnp.asarray(hexp), jnp.asarray(densel))


def kernel(x, fe_fc_w, fe_fc_b, fe_ln_g, fe_ln_b, rgcn_w0, rgcn_root0,
           rgcn_b0, tc_qw0, tc_qb0, tc_kw0, tc_kb0, tc_vw0, tc_vb0, tc_sw0,
           tc_sb0, bn_g0, bn_b0, rgcn_w1, rgcn_root1, rgcn_b1, tc_qw1,
           tc_qb1, tc_kw1, tc_kb1, tc_vw1, tc_vb1, tc_sw1, tc_sb1, bn_g1,
           bn_b1):
    b, n, hs, ws, c = x.shape
    fdim = hs * ws * c
    blk = 1000 if (n % 1000 == 0 and n > 1000) else n
    nb = n // blk
    steps = 3 * nb + 1

    row = lambda a: a.reshape(1, -1)
    abias, inv, hsel_s, hexp, densel = _graph_consts(n)

    cats = []
    for (w, root, rb, qw, qb, kw, kb, vw, vb, sw, sb) in [
        (rgcn_w0, rgcn_root0, rgcn_b0, tc_qw0, tc_qb0, tc_kw0, tc_kb0,
         tc_vw0, tc_vb0, tc_sw0, tc_sb0),
        (rgcn_w1, rgcn_root1, rgcn_b1, tc_qw1, tc_qb1, tc_kw1, tc_kb1,
         tc_vw1, tc_vb1, tc_sw1, tc_sb1),
    ]:
        catw = jnp.concatenate([w[0] + root, w[1], w[2], w[3]], axis=0)
        catp = jnp.concatenate([qw, kw, vw, sw], axis=1)
        catpb = jnp.concatenate([qb, kb, vb, sb]).reshape(1, -1)
        cats.append((catw, row(rb), catp, catpb))
    (catw0, rb0, catp0, catpb0), (catw1, rb1, catp1, catpb1) = cats

    def full(shape):
        nd = len(shape)
        return pl.BlockSpec(shape, lambda s, _n=nd: (0,) * _n)

    outs = []
    for bi in range(b):
        x2 = x[bi].reshape(n, fdim)
        out = pl.pallas_call(
            functools.partial(_mega_kernel, n=n, blk=blk, nb=nb),
            grid=(steps,),
            in_specs=[pl.BlockSpec((blk, fdim),
                                   lambda s: (jnp.minimum(s, nb - 1), 0)),
                      full(fe_fc_w.shape), full((1, HH)), full((1, HH)),
                      full((1, HH)),
                      full(catw0.shape), full((1, HH)), full(catp0.shape),
                      full((1, 4 * HH)), full((1, HH)), full((1, HH)),
                      full(catw1.shape), full((1, HH)), full(catp1.shape),
                      full((1, 4 * HH)), full((1, HH)), full((1, HH)),
                      full(inv.shape), full(abias.shape), full(hsel_s.shape),
                      full(hexp.shape), full(densel.shape)],
            out_specs=pl.BlockSpec((blk, HH),
                                   lambda s: (jnp.maximum(s - (2 * nb + 1), 0), 0)),
            out_shape=jax.ShapeDtypeStruct((n, HH), jnp.float32),
            scratch_shapes=[pltpu.VMEM((n + 2 * PAD, HH), jnp.float32),
                            pltpu.VMEM((n + 2 * PAD, HH), jnp.float32),
                            pltpu.VMEM((1, HH), jnp.float32),
                            pltpu.VMEM((1, HH), jnp.float32),
                            pltpu.VMEM((1, HH), jnp.float32),
                            pltpu.VMEM((1, HH), jnp.float32),
                            pltpu.VMEM((blk, 4 * NTAP16), jnp.float32)],
        )(x2, fe_fc_w, row(fe_fc_b), row(fe_ln_g), row(fe_ln_b),
          catw0, rb0, catp0, catpb0, row(bn_g0), row(bn_b0),
          catw1, rb1, catp1, catpb1, row(bn_g1), row(bn_b1),
          inv, abias, hsel_s, hexp, densel)
        outs.append(out[None])
    return jnp.concatenate(outs, axis=0)


# speedup vs baseline: 1.0086x; 1.0086x over previous
import functools

import jax
import jax.numpy as jnp
import numpy as np
from jax.experimental import pallas as pl
from jax.experimental.pallas import tpu as pltpu

N_HEADS, HEAD_DIM = 4, 32
HH = N_HEADS * HEAD_DIM
PAD = 32
PAST_OFF = [1, 2, 3, 4, 5]
FUT_OFF = [-1, -2, -3, -4, -5]
PER_OFF = [-15, 15]
ALL_OFF = [0] + PAST_OFF + FUT_OFF + PER_OFF
NTAP = len(ALL_OFF)
NTAP16 = 16
NEG = -1e30


def _lroll(x, s):
    return pltpu.roll(x, s, axis=1)


def _mega_kernel(x_ref, few_ref, feb_ref, feg_ref, febeta_ref, jmat_ref,
                 catw0_ref, rb0_ref, catp0_ref, catpb0_ref, bg0_ref, bb0_ref,
                 catw1_ref, rb1_ref, catp1_ref, catpb1_ref, bg1_ref, bb1_ref,
                 inv_ref, abias_ref, hsel_ref, hexp_ref, densel_ref,
                 o_ref, s_nf, s_pre, s_sum0, s_sumsq0, s_sum1, s_sumsq1,
                 scr, *, n, blk, nb):
    s = pl.program_id(0)
    ext = blk + 2 * PAD
    mid0, nmid = PAD - 15, blk + 30

    @pl.when(s < nb)
    def _fe():
        n0 = s * blk
        xb = x_ref[...]
        c = few_ref.shape[0]
        feat = (xb[:, :c] + xb[:, c:2 * c] + xb[:, 2 * c:3 * c]
                + xb[:, 3 * c:4 * c]) * 0.25
        nf = jnp.dot(feat, few_ref[...], preferred_element_type=jnp.float32) + feb_ref[...]
        jm = jmat_ref[...]
        mu = jnp.dot(nf, jm, preferred_element_type=jnp.float32)
        exx = jnp.dot(nf * nf, jm, preferred_element_type=jnp.float32)
        var = exx - mu * mu
        y = (nf - mu) * jax.lax.rsqrt(var + 1e-5) * feg_ref[...] + febeta_ref[...]

        @pl.when(s == 0)
        def _zero_pads():
            z = jnp.zeros((PAD, HH), jnp.float32)
            s_nf[0:PAD, :] = z
            s_nf[PAD + n:2 * PAD + n, :] = z
            s_pre[0:PAD, :] = z
            s_pre[PAD + n:2 * PAD + n, :] = z

        s_nf[pl.ds(PAD + n0, blk), :] = y

    def _layer_body(j, xe, catw_ref, rb_ref, catp_ref, catpb_ref,
                    dst, sum_ref, sumsq_ref):
        n0 = j * blk
        x_mid = xe[mid0:mid0 + nmid]
        u1 = xe[mid0 + 1:mid0 + 1 + nmid]
        u2 = xe[mid0 - 1:mid0 - 1 + nmid]
        for o in PAST_OFF[1:]:
            u1 = u1 + xe[mid0 + o:mid0 + o + nmid]
            u2 = u2 + xe[mid0 - o:mid0 - o + nmid]
        u3 = xe[mid0 - 15:mid0 - 15 + nmid] + xe[mid0 + 15:mid0 + 15 + nmid]
        inv = inv_ref[pl.ds(n0 + mid0, nmid), :]
        cat = jnp.concatenate(
            [x_mid, u1 * inv[:, 0:1], u2 * inv[:, 1:2], u3 * inv[:, 2:3]], axis=1)
        h = jnp.dot(cat, catw_ref[...], preferred_element_type=jnp.float32) + rb_ref[...]

        qkvs = jnp.dot(h, catp_ref[...], preferred_element_type=jnp.float32) + catpb_ref[...]
        q = qkvs[15:15 + blk, 0:HH]
        k = qkvs[:, HH:2 * HH]
        v = qkvs[:, 2 * HH:3 * HH]

        hsel = hsel_ref[...]
        def tap(t):
            d = ALL_OFF[t]
            return jnp.dot(q * k[15 + d:15 + d + blk], hsel,
                           preferred_element_type=jnp.float32)
        for tp in range(6):
            scr[:, 8 * tp:8 * tp + 8] = jnp.concatenate(
                [tap(2 * tp), tap(2 * tp + 1)], axis=1)
        z4 = jnp.zeros((blk, 4), jnp.float32)
        scr[:, 48:56] = jnp.concatenate([tap(12), z4], axis=1)
        scr[:, 56:64] = jnp.zeros((blk, 8), jnp.float32)

        a64 = scr[...] + abias_ref[pl.ds(n0, blk), :]
        m = jnp.maximum(a64, _lroll(a64, 32))
        m = jnp.maximum(m, _lroll(m, 16))
        m = jnp.maximum(m, _lroll(m, 8))
        m = jnp.maximum(m, _lroll(m, 4))
        ex = jnp.exp2(a64 - m)
        denb = jnp.dot(ex, densel_ref[...], preferred_element_type=jnp.float32)

        hexp = hexp_ref[...]
        num = jnp.zeros((blk, HH), jnp.float32)
        for t, d in enumerate(ALL_OFF):
            exb = jnp.dot(ex[:, 4 * t:4 * t + 4], hexp,
                          preferred_element_type=jnp.float32)
            num = num + exb * v[15 + d:15 + d + blk]

        out = num / jnp.maximum(denb, 1e-16) + qkvs[15:15 + blk, 3 * HH:4 * HH]
        dst[pl.ds(PAD + n0, blk), :] = out
        ps = jnp.sum(out, axis=0, keepdims=True)
        pq = jnp.sum(out * out, axis=0, keepdims=True)

        @pl.when(j == 0)
        def _init_stats():
            sum_ref[...] = ps
            sumsq_ref[...] = pq

        @pl.when(j > 0)
        def _acc_stats():
            sum_ref[...] = sum_ref[...] + ps
            sumsq_ref[...] = sumsq_ref[...] + pq

    def _bn(xw, sum_ref, sumsq_ref, bg_ref, bb_ref):
        mu = sum_ref[...] * (1.0 / n)
        var = sumsq_ref[...] * (1.0 / n) - mu * mu
        y = (xw - mu) * jax.lax.rsqrt(var + 1e-5) * bg_ref[...] + bb_ref[...]
        return jnp.where(y >= 0.0, y, 0.01 * y)

    @pl.when(jnp.logical_and(s >= 1, s <= nb))
    def _layer0():
        j = s - 1
        xe = s_nf[pl.ds(j * blk, ext), :]
        _layer_body(j, xe, catw0_ref, rb0_ref, catp0_ref, catpb0_ref,
                    s_pre, s_sum0, s_sumsq0)

    @pl.when(jnp.logical_and(s >= nb + 1, s <= 2 * nb))
    def _layer1():
        j = s - nb - 1
        n0 = j * blk
        raw = s_pre[pl.ds(n0, ext), :]
        vmask = inv_ref[pl.ds(n0, ext), 3:4]
        xe = _bn(raw, s_sum0, s_sumsq0, bg0_ref, bb0_ref) * vmask
        _layer_body(j, xe, catw1_ref, rb1_ref, catp1_ref, catpb1_ref,
                    s_nf, s_sum1, s_sumsq1)

    @pl.when(s >= 2 * nb + 1)
    def _bn1():
        j = s - 2 * nb - 1
        xb = s_nf[pl.ds(PAD + j * blk, blk), :]
        o_ref[...] = _bn(xb, s_sum1, s_sumsq1, bg1_ref, bb1_ref)


def _graph_consts(n):
    g = np.arange(n)
    deltas = np.asarray(ALL_OFF)
    valid = (g[:, None] + deltas[None, :] >= 0) & (g[:, None] + deltas[None, :] < n)
    abias13 = np.where(valid, 0.0, NEG).astype(np.float32)
    abias = np.full((n, 4 * NTAP16), NEG, np.float32)
    abias[:, :4 * NTAP] = np.repeat(abias13, 4, axis=1)

    gp = np.arange(-PAD, n + PAD).astype(np.float32)
    inv = np.zeros((n + 2 * PAD, 4), np.float32)
    inv[:, 0] = 1.0 / np.maximum(np.minimum(5.0, (n - 1) - gp), 1.0)
    inv[:, 1] = 1.0 / np.maximum(np.minimum(5.0, gp), 1.0)
    inv[:, 2] = 1.0 / np.maximum((gp >= 15).astype(np.float32)
                                 + (gp <= n - 16).astype(np.float32), 1.0)
    inv[:, 3] = ((gp >= 0) & (gp <= n - 1)).astype(np.float32)

    hsel = np.repeat(np.eye(N_HEADS, dtype=np.float32), HEAD_DIM, axis=0)
    hsel_s = hsel * np.float32(np.log2(np.e) / np.sqrt(HEAD_DIM))
    hexp = hsel.T
    lanes = np.arange(4 * NTAP16)
    densel = (lanes[:, None] % 4 == np.arange(HH)[None, :] // HEAD_DIM).astype(np.float32)
    jmat = np.full((HH, HH), 1.0 / HH, np.float32)
    return (jnp.asarray(abias), jnp.asarray(inv), jnp.asarray(hsel_s),
            jnp.asarray(hexp), jnp.asarray(densel), jnp.asarray(jmat))


def kernel(x, fe_fc_w, fe_fc_b, fe_ln_g, fe_ln_b, rgcn_w0, rgcn_root0,
           rgcn_b0, tc_qw0, tc_qb0, tc_kw0, tc_kb0, tc_vw0, tc_vb0, tc_sw0,
           tc_sb0, bn_g0, bn_b0, rgcn_w1, rgcn_root1, rgcn_b1, tc_qw1,
           tc_qb1, tc_kw1, tc_kb1, tc_vw1, tc_vb1, tc_sw1, tc_sb1, bn_g1,
           bn_b1):
    b, n, hs, ws, c = x.shape
    fdim = hs * ws * c
    blk = 2000 if (n % 2000 == 0 and n > 2000) else n
    nb = n // blk
    steps = 3 * nb + 1

    row = lambda a: a.reshape(1, -1)
    abias, inv, hsel_s, hexp, densel, jmat = _graph_consts(n)

    cats = []
    for (w, root, rb, qw, qb, kw, kb, vw, vb, sw, sb) in [
        (rgcn_w0, rgcn_root0, rgcn_b0, tc_qw0, tc_qb0, tc_kw0, tc_kb0,
         tc_vw0, tc_vb0, tc_sw0, tc_sb0),
        (rgcn_w1, rgcn_root1, rgcn_b1, tc_qw1, tc_qb1, tc_kw1, tc_kb1,
         tc_vw1, tc_vb1, tc_sw1, tc_sb1),
    ]:
        catw = jnp.concatenate([w[0] + root, w[1], w[2], w[3]], axis=0)
        catp = jnp.concatenate([qw, kw, vw, sw], axis=1)
        catpb = jnp.concatenate([qb, kb, vb, sb]).reshape(1, -1)
        cats.append((catw, row(rb), catp, catpb))
    (catw0, rb0, catp0, catpb0), (catw1, rb1, catp1, catpb1) = cats

    def full(shape):
        nd = len(shape)
        return pl.BlockSpec(shape, lambda s, _n=nd: (0,) * _n)

    outs = []
    for bi in range(b):
        x2 = x[bi].reshape(n, fdim)
        out = pl.pallas_call(
            functools.partial(_mega_kernel, n=n, blk=blk, nb=nb),
            grid=(steps,),
            in_specs=[pl.BlockSpec((blk, fdim),
                                   lambda s: (jnp.minimum(s, nb - 1), 0)),
                      full(fe_fc_w.shape), full((1, HH)), full((1, HH)),
                      full((1, HH)), full(jmat.shape),
                      full(catw0.shape), full((1, HH)), full(catp0.shape),
                      full((1, 4 * HH)), full((1, HH)), full((1, HH)),
                      full(catw1.shape), full((1, HH)), full(catp1.shape),
                      full((1, 4 * HH)), full((1, HH)), full((1, HH)),
                      full(inv.shape), full(abias.shape), full(hsel_s.shape),
                      full(hexp.shape), full(densel.shape)],
            out_specs=pl.BlockSpec((blk, HH),
                                   lambda s: (jnp.maximum(s - (2 * nb + 1), 0), 0)),
            out_shape=jax.ShapeDtypeStruct((n, HH), jnp.float32),
            scratch_shapes=[pltpu.VMEM((n + 2 * PAD, HH), jnp.float32),
                            pltpu.VMEM((n + 2 * PAD, HH), jnp.float32),
                            pltpu.VMEM((1, HH), jnp.float32),
                            pltpu.VMEM((1, HH), jnp.float32),
                            pltpu.VMEM((1, HH), jnp.float32),
                            pltpu.VMEM((1, HH), jnp.float32),
                            pltpu.VMEM((blk, 4 * NTAP16), jnp.float32)],
        )(x2, fe_fc_w, row(fe_fc_b), row(fe_ln_g), row(fe_ln_b), jmat,
          catw0, rb0, catp0, catpb0, row(bn_g0), row(bn_b0),
          catw1, rb1, catp1, catpb1, row(bn_g1), row(bn_b1),
          inv, abias, hsel_s, hexp, densel)
        outs.append(out[None])
    return jnp.concatenate(outs, axis=0)


# speedup vs baseline: 1.0713x; 1.0621x over previous
import functools

import jax
import jax.numpy as jnp
import numpy as np
from jax.experimental import pallas as pl
from jax.experimental.pallas import tpu as pltpu

N_HEADS, HEAD_DIM = 4, 32
HH = N_HEADS * HEAD_DIM
PAD = 32
PAST_OFF = [1, 2, 3, 4, 5]
FUT_OFF = [-1, -2, -3, -4, -5]
PER_OFF = [-15, 15]
ALL_OFF = [0] + PAST_OFF + FUT_OFF + PER_OFF
NTAP = len(ALL_OFF)
NTAP16 = 16
NEG = -1e30


def _lroll(x, s):
    return jnp.concatenate([x[:, s:], x[:, :s]], axis=1)


def _mega_kernel(x_ref, few_ref, feb_ref, feg_ref, febeta_ref,
                 catw0_ref, rb0_ref, catp0_ref, catpb0_ref, bg0_ref, bb0_ref,
                 catw1_ref, rb1_ref, catp1_ref, catpb1_ref, bg1_ref, bb1_ref,
                 inv_ref, abias_ref, hsel_ref, hexp_ref, densel_ref,
                 o_ref, s_nf, s_pre, s_sum0, s_sumsq0, s_sum1, s_sumsq1,
                 scr, *, n, blk, nb):
    s = pl.program_id(0)
    ext = blk + 2 * PAD
    mid0, nmid = PAD - 15, blk + 30

    @pl.when(s < nb)
    def _fe():
        n0 = s * blk
        xb = x_ref[...]
        c = few_ref.shape[0]
        feat = (xb[:, :c] + xb[:, c:2 * c] + xb[:, 2 * c:3 * c]
                + xb[:, 3 * c:4 * c]) * 0.25
        nf = jnp.dot(feat, few_ref[...], preferred_element_type=jnp.float32) + feb_ref[...]
        mu = jnp.mean(nf, axis=-1, keepdims=True)
        var = jnp.mean((nf - mu) ** 2, axis=-1, keepdims=True)
        y = (nf - mu) * jax.lax.rsqrt(var + 1e-5) * feg_ref[...] + febeta_ref[...]

        @pl.when(s == 0)
        def _zero_pads():
            z = jnp.zeros((PAD, HH), jnp.float32)
            s_nf[0:PAD, :] = z
            s_nf[PAD + n:2 * PAD + n, :] = z
            s_pre[0:PAD, :] = z
            s_pre[PAD + n:2 * PAD + n, :] = z

        s_nf[pl.ds(PAD + n0, blk), :] = y

    def _layer_body(j, xe, catw_ref, rb_ref, catp_ref, catpb_ref,
                    dst, sum_ref, sumsq_ref):
        n0 = j * blk
        x_mid = xe[mid0:mid0 + nmid]
        u1 = xe[mid0 + 1:mid0 + 1 + nmid]
        u2 = xe[mid0 - 1:mid0 - 1 + nmid]
        for o in PAST_OFF[1:]:
            u1 = u1 + xe[mid0 + o:mid0 + o + nmid]
            u2 = u2 + xe[mid0 - o:mid0 - o + nmid]
        u3 = xe[mid0 - 15:mid0 - 15 + nmid] + xe[mid0 + 15:mid0 + 15 + nmid]
        inv = inv_ref[pl.ds(n0 + mid0, nmid), :]
        cat = jnp.concatenate(
            [x_mid, u1 * inv[:, 0:1], u2 * inv[:, 1:2], u3 * inv[:, 2:3]], axis=1)
        h = jnp.dot(cat, catw_ref[...], preferred_element_type=jnp.float32) + rb_ref[...]

        qkvs = jnp.dot(h, catp_ref[...], preferred_element_type=jnp.float32) + catpb_ref[...]
        q = qkvs[15:15 + blk, 0:HH]
        k = qkvs[:, HH:2 * HH]
        v = qkvs[:, 2 * HH:3 * HH]

        hsel = hsel_ref[...]
        def tap(t):
            d = ALL_OFF[t]
            return jnp.dot(q * k[15 + d:15 + d + blk], hsel,
                           preferred_element_type=jnp.float32)
        for tp in range(6):
            scr[:, 8 * tp:8 * tp + 8] = jnp.concatenate(
                [tap(2 * tp), tap(2 * tp + 1)], axis=1)
        z4 = jnp.zeros((blk, 4), jnp.float32)
        scr[:, 48:56] = jnp.concatenate([tap(12), z4], axis=1)
        scr[:, 56:64] = jnp.zeros((blk, 8), jnp.float32)

        a64 = scr[...] + abias_ref[pl.ds(n0, blk), :]
        m = jnp.maximum(a64, _lroll(a64, 32))
        m = jnp.maximum(m, _lroll(m, 16))
        m = jnp.maximum(m, _lroll(m, 8))
        m = jnp.maximum(m, _lroll(m, 4))
        ex = jnp.exp2(a64 - m)
        denb = jnp.dot(ex, densel_ref[...], preferred_element_type=jnp.float32)

        hexp = hexp_ref[...]
        num = jnp.zeros((blk, HH), jnp.float32)
        for t, d in enumerate(ALL_OFF):
            exb = jnp.dot(ex[:, 4 * t:4 * t + 4], hexp,
                          preferred_element_type=jnp.float32)
            num = num + exb * v[15 + d:15 + d + blk]

        out = num / jnp.maximum(denb, 1e-16) + qkvs[15:15 + blk, 3 * HH:4 * HH]
        dst[pl.ds(PAD + n0, blk), :] = out
        ps = jnp.sum(out, axis=0, keepdims=True)
        pq = jnp.sum(out * out, axis=0, keepdims=True)

        @pl.when(j == 0)
        def _init_stats():
            sum_ref[...] = ps
            sumsq_ref[...] = pq

        @pl.when(j > 0)
        def _acc_stats():
            sum_ref[...] = sum_ref[...] + ps
            sumsq_ref[...] = sumsq_ref[...] + pq

    def _bn(xw, sum_ref, sumsq_ref, bg_ref, bb_ref):
        mu = sum_ref[...] * (1.0 / n)
        var = sumsq_ref[...] * (1.0 / n) - mu * mu
        y = (xw - mu) * jax.lax.rsqrt(var + 1e-5) * bg_ref[...] + bb_ref[...]
        return jnp.where(y >= 0.0, y, 0.01 * y)

    @pl.when(jnp.logical_and(s >= 1, s <= nb))
    def _layer0():
        j = s - 1
        xe = s_nf[pl.ds(j * blk, ext), :]
        _layer_body(j, xe, catw0_ref, rb0_ref, catp0_ref, catpb0_ref,
                    s_pre, s_sum0, s_sumsq0)

    @pl.when(jnp.logical_and(s >= nb + 1, s <= 2 * nb))
    def _layer1():
        j = s - nb - 1
        n0 = j * blk
        raw = s_pre[pl.ds(n0, ext), :]
        vmask = inv_ref[pl.ds(n0, ext), 3:4]
        xe = _bn(raw, s_sum0, s_sumsq0, bg0_ref, bb0_ref) * vmask
        _layer_body(j, xe, catw1_ref, rb1_ref, catp1_ref, catpb1_ref,
                    s_nf, s_sum1, s_sumsq1)

    @pl.when(s >= 2 * nb + 1)
    def _bn1():
        j = s - 2 * nb - 1
        xb = s_nf[pl.ds(PAD + j * blk, blk), :]
        o_ref[...] = _bn(xb, s_sum1, s_sumsq1, bg1_ref, bb1_ref)


def _graph_consts(n):
    g = np.arange(n)
    deltas = np.asarray(ALL_OFF)
    valid = (g[:, None] + deltas[None, :] >= 0) & (g[:, None] + deltas[None, :] < n)
    abias13 = np.where(valid, 0.0, NEG).astype(np.float32)
    abias = np.full((n, 4 * NTAP16), NEG, np.float32)
    abias[:, :4 * NTAP] = np.repeat(abias13, 4, axis=1)

    gp = np.arange(-PAD, n + PAD).astype(np.float32)
    inv = np.zeros((n + 2 * PAD, 4), np.float32)
    inv[:, 0] = 1.0 / np.maximum(np.minimum(5.0, (n - 1) - gp), 1.0)
    inv[:, 1] = 1.0 / np.maximum(np.minimum(5.0, gp), 1.0)
    inv[:, 2] = 1.0 / np.maximum((gp >= 15).astype(np.float32)
                                 + (gp <= n - 16).astype(np.float32), 1.0)
    inv[:, 3] = ((gp >= 0) & (gp <= n - 1)).astype(np.float32)

    hsel = np.repeat(np.eye(N_HEADS, dtype=np.float32), HEAD_DIM, axis=0)
    hsel_s = hsel * np.float32(np.log2(np.e) / np.sqrt(HEAD_DIM))
    hexp = hsel.T
    lanes = np.arange(4 * NTAP16)
    densel = (lanes[:, None] % 4 == np.arange(HH)[None, :] // HEAD_DIM).astype(np.float32)
    return (jnp.asarray(abias), jnp.asarray(inv), jnp.asarray(hsel_s),
            jnp.asarray(hexp), jnp.asarray(densel))


def kernel(x, fe_fc_w, fe_fc_b, fe_ln_g, fe_ln_b, rgcn_w0, rgcn_root0,
           rgcn_b0, tc_qw0, tc_qb0, tc_kw0, tc_kb0, tc_vw0, tc_vb0, tc_sw0,
           tc_sb0, bn_g0, bn_b0, rgcn_w1, rgcn_root1, rgcn_b1, tc_qw1,
           tc_qb1, tc_kw1, tc_kb1, tc_vw1, tc_vb1, tc_sw1, tc_sb1, bn_g1,
           bn_b1):
    b, n, hs, ws, c = x.shape
    fdim = hs * ws * c
    blk = 2000 if (n % 2000 == 0 and n > 2000) else n
    nb = n // blk
    steps = 3 * nb + 1

    row = lambda a: a.reshape(1, -1)
    abias, inv, hsel_s, hexp, densel = _graph_consts(n)

    cats = []
    for (w, root, rb, qw, qb, kw, kb, vw, vb, sw, sb) in [
        (rgcn_w0, rgcn_root0, rgcn_b0, tc_qw0, tc_qb0, tc_kw0, tc_kb0,
         tc_vw0, tc_vb0, tc_sw0, tc_sb0),
        (rgcn_w1, rgcn_root1, rgcn_b1, tc_qw1, tc_qb1, tc_kw1, tc_kb1,
         tc_vw1, tc_vb1, tc_sw1, tc_sb1),
    ]:
        catw = jnp.concatenate([w[0] + root, w[1], w[2], w[3]], axis=0)
        catp = jnp.concatenate([qw, kw, vw, sw], axis=1)
        catpb = jnp.concatenate([qb, kb, vb, sb]).reshape(1, -1)
        cats.append((catw, row(rb), catp, catpb))
    (catw0, rb0, catp0, catpb0), (catw1, rb1, catp1, catpb1) = cats

    def full(shape):
        nd = len(shape)
        return pl.BlockSpec(shape, lambda s, _n=nd: (0,) * _n)

    outs = []
    for bi in range(b):
        x2 = x[bi].reshape(n, fdim)
        out = pl.pallas_call(
            functools.partial(_mega_kernel, n=n, blk=blk, nb=nb),
            grid=(steps,),
            in_specs=[pl.BlockSpec((blk, fdim),
                                   lambda s: (jnp.minimum(s, nb - 1), 0)),
                      full(fe_fc_w.shape), full((1, HH)), full((1, HH)),
                      full((1, HH)),
                      full(catw0.shape), full((1, HH)), full(catp0.shape),
                      full((1, 4 * HH)), full((1, HH)), full((1, HH)),
                      full(catw1.shape), full((1, HH)), full(catp1.shape),
                      full((1, 4 * HH)), full((1, HH)), full((1, HH)),
                      full(inv.shape), full(abias.shape), full(hsel_s.shape),
                      full(hexp.shape), full(densel.shape)],
            out_specs=pl.BlockSpec((blk, HH),
                                   lambda s: (jnp.maximum(s - (2 * nb + 1), 0), 0)),
            out_shape=jax.ShapeDtypeStruct((n, HH), jnp.float32),
            scratch_shapes=[pltpu.VMEM((n + 2 * PAD, HH), jnp.float32),
                            pltpu.VMEM((n + 2 * PAD, HH), jnp.float32),
                            pltpu.VMEM((1, HH), jnp.float32),
                            pltpu.VMEM((1, HH), jnp.float32),
                            pltpu.VMEM((1, HH), jnp.float32),
                            pltpu.VMEM((1, HH), jnp.float32),
                            pltpu.VMEM((blk, 4 * NTAP16), jnp.float32)],
        )(x2, fe_fc_w, row(fe_fc_b), row(fe_ln_g), row(fe_ln_b),
          catw0, rb0, catp0, catpb0, row(bn_g0), row(bn_b0),
          catw1, rb1, catp1, catpb1, row(bn_g1), row(bn_b1),
          inv, abias, hsel_s, hexp, densel)
        outs.append(out[None])
    return jnp.concatenate(outs, axis=0)


# speedup vs baseline: 1.0725x; 1.0011x over previous
import functools

import jax
import jax.numpy as jnp
import numpy as np
from jax.experimental import pallas as pl
from jax.experimental.pallas import tpu as pltpu

N_HEADS, HEAD_DIM = 4, 32
HH = N_HEADS * HEAD_DIM
PAD = 32
PAST_OFF = [1, 2, 3, 4, 5]
FUT_OFF = [-1, -2, -3, -4, -5]
PER_OFF = [-15, 15]
ALL_OFF = [0] + PAST_OFF + FUT_OFF + PER_OFF
NTAP = len(ALL_OFF)
NTAP16 = 16
NEG = -1e30


def _lroll(x, s):
    return jnp.concatenate([x[:, s:], x[:, :s]], axis=1)


def _mega_kernel(x_ref, few_ref, feb_ref, feg_ref, febeta_ref,
                 catw0_ref, rb0_ref, catp0_ref, catpb0_ref, bg0_ref, bb0_ref,
                 catw1_ref, rb1_ref, catp1_ref, catpb1_ref, bg1_ref, bb1_ref,
                 inv_ref, abias_ref, hsel_ref, hexp_ref, densel_ref,
                 o_ref, s_nf, s_pre, s_sum0, s_sumsq0, s_sum1, s_sumsq1,
                 scr, *, n, blk, nb):
    s = pl.program_id(0)
    ext = blk + 2 * PAD
    mid0, nmid = PAD - 15, blk + 30

    @pl.when(s < nb)
    def _fe():
        n0 = s * blk
        xb = x_ref[...]
        c = few_ref.shape[0]
        feat = (xb[:, :c] + xb[:, c:2 * c] + xb[:, 2 * c:3 * c]
                + xb[:, 3 * c:4 * c]) * 0.25
        nf = jnp.dot(feat, few_ref[...], preferred_element_type=jnp.float32) + feb_ref[...]
        mu = jnp.mean(nf, axis=-1, keepdims=True)
        var = jnp.mean((nf - mu) ** 2, axis=-1, keepdims=True)
        y = (nf - mu) * jax.lax.rsqrt(var + 1e-5) * feg_ref[...] + febeta_ref[...]

        @pl.when(s == 0)
        def _zero_pads():
            z = jnp.zeros((PAD, HH), jnp.float32)
            s_nf[0:PAD, :] = z
            s_nf[PAD + n:2 * PAD + n, :] = z
            s_pre[0:PAD, :] = z
            s_pre[PAD + n:2 * PAD + n, :] = z

        s_nf[pl.ds(PAD + n0, blk), :] = y

    def _layer_body(j, xe, catw_ref, rb_ref, catp_ref, catpb_ref,
                    dst, sum_ref, sumsq_ref):
        n0 = j * blk
        x_mid = xe[mid0:mid0 + nmid]
        u1 = xe[mid0 + 1:mid0 + 1 + nmid]
        u2 = xe[mid0 - 1:mid0 - 1 + nmid]
        for o in PAST_OFF[1:]:
            u1 = u1 + xe[mid0 + o:mid0 + o + nmid]
            u2 = u2 + xe[mid0 - o:mid0 - o + nmid]
        u3 = xe[mid0 - 15:mid0 - 15 + nmid] + xe[mid0 + 15:mid0 + 15 + nmid]
        inv = inv_ref[pl.ds(n0 + mid0, nmid), :]
        cat = jnp.concatenate(
            [x_mid, u1 * inv[:, 0:1], u2 * inv[:, 1:2], u3 * inv[:, 2:3]], axis=1)
        h = jnp.dot(cat, catw_ref[...], preferred_element_type=jnp.float32) + rb_ref[...]

        qkvs = jnp.dot(h, catp_ref[...], preferred_element_type=jnp.float32) + catpb_ref[...]
        q = qkvs[15:15 + blk, 0:HH]
        k = qkvs[:, HH:2 * HH]
        v = qkvs[:, 2 * HH:3 * HH]

        hsel = hsel_ref[...]
        def tap(t):
            d = ALL_OFF[t]
            return jnp.dot(q * k[15 + d:15 + d + blk], hsel,
                           preferred_element_type=jnp.float32)
        for tp in range(6):
            scr[:, 8 * tp:8 * tp + 8] = jnp.concatenate(
                [tap(2 * tp), tap(2 * tp + 1)], axis=1)
        z4 = jnp.zeros((blk, 4), jnp.float32)
        scr[:, 48:56] = jnp.concatenate([tap(12), z4], axis=1)
        scr[:, 56:64] = jnp.zeros((blk, 8), jnp.float32)

        a64 = scr[...] + abias_ref[pl.ds(n0, blk), :]
        m = jnp.maximum(a64, _lroll(a64, 32))
        m = jnp.maximum(m, _lroll(m, 16))
        m = jnp.maximum(m, _lroll(m, 8))
        m = jnp.maximum(m, _lroll(m, 4))
        ex = jnp.exp2(a64 - m)
        denb = jnp.dot(ex, densel_ref[...], preferred_element_type=jnp.float32)

        hexp = hexp_ref[...]
        num = jnp.zeros((blk, HH), jnp.float32)
        for t, d in enumerate(ALL_OFF):
            exb = jnp.dot(ex[:, 4 * t:4 * t + 4], hexp,
                          preferred_element_type=jnp.float32)
            num = num + exb * v[15 + d:15 + d + blk]

        out = num / jnp.maximum(denb, 1e-16) + qkvs[15:15 + blk, 3 * HH:4 * HH]
        dst[pl.ds(PAD + n0, blk), :] = out
        ps = jnp.sum(out, axis=0, keepdims=True)
        pq = jnp.sum(out * out, axis=0, keepdims=True)

        @pl.when(j == 0)
        def _init_stats():
            sum_ref[...] = ps
            sumsq_ref[...] = pq

        @pl.when(j > 0)
        def _acc_stats():
            sum_ref[...] = sum_ref[...] + ps
            sumsq_ref[...] = sumsq_ref[...] + pq

    def _bn(xw, sum_ref, sumsq_ref, bg_ref, bb_ref):
        mu = sum_ref[...] * (1.0 / n)
        var = sumsq_ref[...] * (1.0 / n) - mu * mu
        y = (xw - mu) * jax.lax.rsqrt(var + 1e-5) * bg_ref[...] + bb_ref[...]
        return jnp.where(y >= 0.0, y, 0.01 * y)

    @pl.when(jnp.logical_and(s >= 1, s <= nb))
    def _layer0():
        j = s - 1
        xe = s_nf[pl.ds(j * blk, ext), :]
        _layer_body(j, xe, catw0_ref, rb0_ref, catp0_ref, catpb0_ref,
                    s_pre, s_sum0, s_sumsq0)

    @pl.when(jnp.logical_and(s >= nb + 1, s <= 2 * nb))
    def _layer1():
        j = s - nb - 1
        n0 = j * blk
        raw = s_pre[pl.ds(n0, ext), :]
        vmask = inv_ref[pl.ds(n0, ext), 3:4]
        xe = _bn(raw, s_sum0, s_sumsq0, bg0_ref, bb0_ref) * vmask
        _layer_body(j, xe, catw1_ref, rb1_ref, catp1_ref, catpb1_ref,
                    s_nf, s_sum1, s_sumsq1)

    @pl.when(s >= 2 * nb + 1)
    def _bn1():
        xb = s_nf[PAD:PAD + n, :]
        o_ref[...] = _bn(xb, s_sum1, s_sumsq1, bg1_ref, bb1_ref)


def _graph_consts(n):
    g = np.arange(n)
    deltas = np.asarray(ALL_OFF)
    valid = (g[:, None] + deltas[None, :] >= 0) & (g[:, None] + deltas[None, :] < n)
    abias13 = np.where(valid, 0.0, NEG).astype(np.float32)
    abias = np.full((n, 4 * NTAP16), NEG, np.float32)
    abias[:, :4 * NTAP] = np.repeat(abias13, 4, axis=1)

    gp = np.arange(-PAD, n + PAD).astype(np.float32)
    inv = np.zeros((n + 2 * PAD, 4), np.float32)
    inv[:, 0] = 1.0 / np.maximum(np.minimum(5.0, (n - 1) - gp), 1.0)
    inv[:, 1] = 1.0 / np.maximum(np.minimum(5.0, gp), 1.0)
    inv[:, 2] = 1.0 / np.maximum((gp >= 15).astype(np.float32)
                                 + (gp <= n - 16).astype(np.float32), 1.0)
    inv[:, 3] = ((gp >= 0) & (gp <= n - 1)).astype(np.float32)

    hsel = np.repeat(np.eye(N_HEADS, dtype=np.float32), HEAD_DIM, axis=0)
    hsel_s = hsel * np.float32(np.log2(np.e) / np.sqrt(HEAD_DIM))
    hexp = hsel.T
    lanes = np.arange(4 * NTAP16)
    densel = (lanes[:, None] % 4 == np.arange(HH)[None, :] // HEAD_DIM).astype(np.float32)
    return (jnp.asarray(abias), jnp.asarray(inv), jnp.asarray(hsel_s),
            jnp.asarray(hexp), jnp.asarray(densel))


def kernel(x, fe_fc_w, fe_fc_b, fe_ln_g, fe_ln_b, rgcn_w0, rgcn_root0,
           rgcn_b0, tc_qw0, tc_qb0, tc_kw0, tc_kb0, tc_vw0, tc_vb0, tc_sw0,
           tc_sb0, bn_g0, bn_b0, rgcn_w1, rgcn_root1, rgcn_b1, tc_qw1,
           tc_qb1, tc_kw1, tc_kb1, tc_vw1, tc_vb1, tc_sw1, tc_sb1, bn_g1,
           bn_b1):
    b, n, hs, ws, c = x.shape
    fdim = hs * ws * c
    blk = 2000 if (n % 2000 == 0 and n > 2000) else n
    nb = n // blk
    steps = 2 * nb + 2

    row = lambda a: a.reshape(1, -1)
    abias, inv, hsel_s, hexp, densel = _graph_consts(n)

    cats = []
    for (w, root, rb, qw, qb, kw, kb, vw, vb, sw, sb) in [
        (rgcn_w0, rgcn_root0, rgcn_b0, tc_qw0, tc_qb0, tc_kw0, tc_kb0,
         tc_vw0, tc_vb0, tc_sw0, tc_sb0),
        (rgcn_w1, rgcn_root1, rgcn_b1, tc_qw1, tc_qb1, tc_kw1, tc_kb1,
         tc_vw1, tc_vb1, tc_sw1, tc_sb1),
    ]:
        catw = jnp.concatenate([w[0] + root, w[1], w[2], w[3]], axis=0)
        catp = jnp.concatenate([qw, kw, vw, sw], axis=1)
        catpb = jnp.concatenate([qb, kb, vb, sb]).reshape(1, -1)
        cats.append((catw, row(rb), catp, catpb))
    (catw0, rb0, catp0, catpb0), (catw1, rb1, catp1, catpb1) = cats

    def full(shape):
        nd = len(shape)
        return pl.BlockSpec(shape, lambda s, _n=nd: (0,) * _n)

    outs = []
    for bi in range(b):
        x2 = x[bi].reshape(n, fdim)
        out = pl.pallas_call(
            functools.partial(_mega_kernel, n=n, blk=blk, nb=nb),
            grid=(steps,),
            in_specs=[pl.BlockSpec((blk, fdim),
                                   lambda s: (jnp.minimum(s, nb - 1), 0)),
                      full(fe_fc_w.shape), full((1, HH)), full((1, HH)),
                      full((1, HH)),
                      full(catw0.shape), full((1, HH)), full(catp0.shape),
                      full((1, 4 * HH)), full((1, HH)), full((1, HH)),
                      full(catw1.shape), full((1, HH)), full(catp1.shape),
                      full((1, 4 * HH)), full((1, HH)), full((1, HH)),
                      full(inv.shape), full(abias.shape), full(hsel_s.shape),
                      full(hexp.shape), full(densel.shape)],
            out_specs=pl.BlockSpec((n, HH), lambda s: (0, 0)),
            out_shape=jax.ShapeDtypeStruct((n, HH), jnp.float32),
            scratch_shapes=[pltpu.VMEM((n + 2 * PAD, HH), jnp.float32),
                            pltpu.VMEM((n + 2 * PAD, HH), jnp.float32),
                            pltpu.VMEM((1, HH), jnp.float32),
                            pltpu.VMEM((1, HH), jnp.float32),
                            pltpu.VMEM((1, HH), jnp.float32),
                            pltpu.VMEM((1, HH), jnp.float32),
                            pltpu.VMEM((blk, 4 * NTAP16), jnp.float32)],
        )(x2, fe_fc_w, row(fe_fc_b), row(fe_ln_g), row(fe_ln_b),
          catw0, rb0, catp0, catpb0, row(bn_g0), row(bn_b0),
          catw1, rb1, catp1, catpb1, row(bn_g1), row(bn_b1),
          inv, abias, hsel_s, hexp, densel)
        outs.append(out[None])
    return jnp.concatenate(outs, axis=0)
